# baseline (speedup 1.0000x reference)
import jax
import jax.numpy as jnp
from jax.experimental import pallas as pl
from jax.experimental.pallas import tpu as pltpu

B, C, H, W = 32, 768, 24, 24


def _body(idx_ref, x_ref, e_ref, o_ref):
    o_ref[...] = x_ref[...] + e_ref[...]


def kernel(x, anatomy_idx, emb_table):
    emb4 = emb_table[:, :, None, None]
    return pl.pallas_call(
        _body,
        grid_spec=pltpu.PrefetchScalarGridSpec(
            num_scalar_prefetch=1,
            grid=(B,),
            in_specs=[
                pl.BlockSpec((1, C, H, W), lambda b, idx: (b, 0, 0, 0)),
                pl.BlockSpec((1, C, 1, 1), lambda b, idx: (idx[b], 0, 0, 0)),
            ],
            out_specs=pl.BlockSpec((1, C, H, W), lambda b, idx: (b, 0, 0, 0)),
        ),
        out_shape=jax.ShapeDtypeStruct((B, C, H, W), jnp.float32),
    )(anatomy_idx.astype(jnp.int32), x, emb4)


# speedup vs baseline: 3.3995x; 3.3995x over previous
import jax
import jax.numpy as jnp
from jax.experimental import pallas as pl
from jax.experimental.pallas import tpu as pltpu

B, C, H, W = 32, 768, 24, 24
HW = H * W


def _body(idx_ref, x_ref, e_ref, o_ref):
    o_ref[...] = x_ref[...] + e_ref[...]


def kernel(x, anatomy_idx, emb_table):
    x3 = x.reshape(B, C, HW)
    emb3 = emb_table[:, :, None]
    out = pl.pallas_call(
        _body,
        grid_spec=pltpu.PrefetchScalarGridSpec(
            num_scalar_prefetch=1,
            grid=(B,),
            in_specs=[
                pl.BlockSpec((1, C, HW), lambda b, idx: (b, 0, 0)),
                pl.BlockSpec((1, C, 1), lambda b, idx: (idx[b], 0, 0)),
            ],
            out_specs=pl.BlockSpec((1, C, HW), lambda b, idx: (b, 0, 0)),
        ),
        out_shape=jax.ShapeDtypeStruct((B, C, HW), jnp.float32),
    )(anatomy_idx.astype(jnp.int32), x3, emb3)
    return out.reshape(B, C, H, W)
